# baseline (speedup 1.0000x reference)
import functools

import jax
import jax.numpy as jnp
import numpy as np
from jax import lax
from jax.experimental import pallas as pl
from jax.experimental.pallas import tpu as pltpu
from jax.experimental.pallas import tpu_sc as plsc

NC, NS = 2, 16
NW = NC * NS
NQ = 32
D_GATE = 256
CHUNK = 128

_PERM32 = np.arange(32) // 2 + (np.arange(32) % 2) * 16
_PERM256 = (np.arange(256) // 32) * 32 + _PERM32[np.arange(256) % 32]


def _sc_gather_counts(gates_flat, qubits_flat, gate_table, m):
    tpw = m // NW
    nchunk = tpw // CHUNK
    mesh = plsc.VectorSubcoreMesh(core_axis_name="c", subcore_axis_name="s")

    @functools.partial(
        pl.kernel,
        mesh=mesh,
        out_type=(
            jax.ShapeDtypeStruct((m * D_GATE // 2,), jnp.int32),
            jax.ShapeDtypeStruct((m * NQ // 2,), jnp.int32),
        ),
        scratch_types=[
            pltpu.VMEM((CHUNK,), jnp.int32),
            pltpu.VMEM((CHUNK, D_GATE), jnp.float32),
            pltpu.VMEM((CHUNK * D_GATE // 2,), jnp.int32),
            pltpu.VMEM((CHUNK * NQ,), jnp.int32),
            pltpu.VMEM((CHUNK * NQ,), jnp.float32),
            pltpu.VMEM((CHUNK * NQ // 2,), jnp.int32),
            pltpu.SemaphoreType.DMA,
        ],
        compiler_params=pltpu.CompilerParams(needs_layout_passes=False),
    )
    def k(gates_hbm, qubits_hbm, table_hbm, emb_hbm, counts_hbm,
          idx_v, rows_v, rows_bf_v, qub_v, cnt_v, cnt_bf_v, sem):
        wid = lax.axis_index("s") * NC + lax.axis_index("c")
        base0 = wid * tpw

        ones = jnp.ones((16,), jnp.float32)
        zeros = jnp.zeros((16,), jnp.float32)

        def chunk_body(ci, carry):
            base = base0 + ci * CHUNK
            pltpu.sync_copy(gates_hbm.at[pl.ds(base, CHUNK)], idx_v)
            gather = pltpu.async_copy(table_hbm.at[idx_v], rows_v, sem)
            pltpu.sync_copy(qubits_hbm.at[pl.ds(base * NQ, CHUNK * NQ)],
                            qub_v)

            def zero_body(j, c):
                cnt_v[pl.ds(j * 16, 16)] = zeros
                return c
            lax.fori_loop(0, CHUNK * NQ // 16, zero_body, None, unroll=8)

            def tok_body(t, c):
                b = t * NQ
                q0 = qub_v[pl.ds(b, 16)]
                q1 = qub_v[pl.ds(b + 16, 16)]
                plsc.addupdate_scatter(cnt_v, [q0 + b], ones)
                plsc.addupdate_scatter(cnt_v, [q1 + b], ones)
                return c
            lax.fori_loop(0, CHUNK, tok_body, None, unroll=4)

            def cpack_body(j, c):
                a = cnt_v[pl.ds(j * 32, 16)]
                b2 = cnt_v[pl.ds(j * 32 + 16, 16)]
                pk = plsc.pack(a, b2, format=plsc.PackFormat.INTERLEAVED)
                cnt_bf_v[pl.ds(j * 16, 16)] = plsc.bitcast(pk, jnp.int32)
                return c
            lax.fori_loop(0, CHUNK * NQ // 32, cpack_body, None, unroll=8)

            gather.wait()

            def epack_body(t, c):
                for j in range(D_GATE // 32):
                    a = rows_v[t, pl.ds(j * 32, 16)]
                    b2 = rows_v[t, pl.ds(j * 32 + 16, 16)]
                    pk = plsc.pack(a, b2,
                                   format=plsc.PackFormat.INTERLEAVED)
                    rows_bf_v[pl.ds(t * (D_GATE // 2) + j * 16, 16)] = (
                        plsc.bitcast(pk, jnp.int32))
                return c
            lax.fori_loop(0, CHUNK, epack_body, None)

            pltpu.sync_copy(
                rows_bf_v,
                emb_hbm.at[pl.ds(base * (D_GATE // 2),
                                 CHUNK * D_GATE // 2)])
            pltpu.sync_copy(
                cnt_bf_v,
                counts_hbm.at[pl.ds(base * (NQ // 2), CHUNK * NQ // 2)])
            return carry

        lax.fori_loop(0, nchunk, chunk_body, None)

    return k(gates_flat, qubits_flat, gate_table)


def _tc_combine(gate_emb, counts, params, qt_perm, Wp, bp2, Wf, Wfg_perm,
                bf2, m, d_model, tm):
    d4 = d_model // 4

    def body(g_ref, c_ref, p_ref, qt_ref, wp_ref, wf_ref, wfg_ref, bp_ref,
             bf_ref, o_ref):
        wf = wf_ref[...]
        wq2 = jnp.dot(qt_ref[...], wf[D_GATE:D_GATE + d4],
                      preferred_element_type=jnp.float32)
        wp2 = jnp.dot(wp_ref[...], wf[D_GATE + d4:],
                      preferred_element_type=jnp.float32)
        bias = jnp.dot(bp_ref[...], wf[D_GATE + d4:],
                       preferred_element_type=jnp.float32) + bf_ref[...]
        acc = jnp.dot(g_ref[...], wfg_ref[...].astype(jnp.bfloat16),
                      preferred_element_type=jnp.float32)
        acc = acc + jnp.dot(c_ref[...], wq2.astype(jnp.bfloat16),
                            preferred_element_type=jnp.float32) * (1.0 / NQ)
        acc = acc + jnp.dot(p_ref[...], wp2,
                            preferred_element_type=jnp.float32)
        o_ref[...] = acc + bias

    return pl.pallas_call(
        body,
        grid=(m // tm,),
        in_specs=[
            pl.BlockSpec((tm, D_GATE), lambda i: (i, 0)),
            pl.BlockSpec((tm, NQ), lambda i: (i, 0)),
            pl.BlockSpec((tm, 8), lambda i: (i, 0)),
            pl.BlockSpec((NQ, d_model // 4), lambda i: (0, 0)),
            pl.BlockSpec((8, d_model // 4), lambda i: (0, 0)),
            pl.BlockSpec((d_model, d_model), lambda i: (0, 0)),
            pl.BlockSpec((D_GATE, d_model), lambda i: (0, 0)),
            pl.BlockSpec((1, d_model // 4), lambda i: (0, 0)),
            pl.BlockSpec((1, d_model), lambda i: (0, 0)),
        ],
        out_specs=pl.BlockSpec((tm, d_model), lambda i: (i, 0)),
        out_shape=jax.ShapeDtypeStruct((m, d_model), jnp.float32),
    )(gate_emb, counts, params, qt_perm, Wp, Wf, Wfg_perm, bp2, bf2)


def kernel(gates, qubits, parameters, gate_table, qubit_table, Wp, bp, Wf,
           bf):
    b, s = gates.shape
    m = b * s
    d_model = Wf.shape[0]

    gates_flat = gates.reshape(m).astype(jnp.int32)
    qubits_flat = qubits.reshape(m * NQ).astype(jnp.int32)
    params2 = parameters.reshape(m, parameters.shape[-1])

    gate_emb_i32, counts_i32 = _sc_gather_counts(gates_flat, qubits_flat,
                                                 gate_table, m)
    gate_emb = lax.bitcast_convert_type(
        gate_emb_i32, jnp.bfloat16).reshape(m, D_GATE)
    counts2 = lax.bitcast_convert_type(
        counts_i32, jnp.bfloat16).reshape(m, NQ)

    qt_perm = qubit_table[_PERM32]
    Wfg_perm = Wf[:D_GATE][_PERM256]

    out = _tc_combine(gate_emb, counts2, params2, qt_perm, Wp,
                      bp.reshape(1, -1), Wf, Wfg_perm, bf.reshape(1, -1),
                      m, d_model, tm=512)
    return out.reshape(b, s, d_model)


# speedup vs baseline: 2.6091x; 2.6091x over previous
import functools

import jax
import jax.numpy as jnp
from jax import lax
from jax.experimental import pallas as pl
from jax.experimental.pallas import tpu as pltpu
from jax.experimental.pallas import tpu_sc as plsc

NC, NS = 2, 16
NW = NC * NS
NQ = 32
D_GATE = 256
CHUNK = 128


def _sc_gather_counts(gates_flat, qubits_flat, gate_table, m):
    tpw = m // NW
    nchunk = tpw // CHUNK
    mesh = plsc.VectorSubcoreMesh(core_axis_name="c", subcore_axis_name="s")

    @functools.partial(
        pl.kernel,
        mesh=mesh,
        out_type=(
            jax.ShapeDtypeStruct((m, D_GATE), jnp.float32),
            jax.ShapeDtypeStruct((m * NQ,), jnp.float32),
        ),
        scratch_types=[
            pltpu.VMEM((CHUNK,), jnp.int32),
            pltpu.VMEM((CHUNK, D_GATE), jnp.float32),
            pltpu.VMEM((CHUNK * NQ,), jnp.int32),
            pltpu.VMEM((CHUNK * NQ,), jnp.float32),
            pltpu.SemaphoreType.DMA,
        ],
        compiler_params=pltpu.CompilerParams(needs_layout_passes=False),
    )
    def k(gates_hbm, qubits_hbm, table_hbm, emb_hbm, counts_hbm,
          idx_v, rows_v, qub_v, cnt_v, sem):
        wid = lax.axis_index("s") * NC + lax.axis_index("c")
        base0 = wid * tpw

        ones = jnp.ones((16,), jnp.float32)
        zeros = jnp.zeros((16,), jnp.float32)

        def chunk_body(ci, carry):
            base = base0 + ci * CHUNK
            pltpu.sync_copy(gates_hbm.at[pl.ds(base, CHUNK)], idx_v)
            gather = pltpu.async_copy(table_hbm.at[idx_v], rows_v, sem)
            pltpu.sync_copy(qubits_hbm.at[pl.ds(base * NQ, CHUNK * NQ)],
                            qub_v)

            def zero_body(j, c):
                cnt_v[pl.ds(j * 16, 16)] = zeros
                return c
            lax.fori_loop(0, CHUNK * NQ // 16, zero_body, None, unroll=8)

            def tok_body(t, c):
                b = t * NQ
                q0 = qub_v[pl.ds(b, 16)]
                q1 = qub_v[pl.ds(b + 16, 16)]
                plsc.addupdate_scatter(cnt_v, [q0 + b], ones)
                plsc.addupdate_scatter(cnt_v, [q1 + b], ones)
                return c
            lax.fori_loop(0, CHUNK, tok_body, None, unroll=4)

            gather.wait()
            pltpu.sync_copy(rows_v, emb_hbm.at[pl.ds(base, CHUNK)])
            pltpu.sync_copy(cnt_v,
                            counts_hbm.at[pl.ds(base * NQ, CHUNK * NQ)])
            return carry

        lax.fori_loop(0, nchunk, chunk_body, None)

    return k(gates_flat, qubits_flat, gate_table)


def _tc_combine(gate_emb, counts, params, qubit_table, Wp, bp2, Wf, bf2,
                m, d_model, tm):
    d4 = d_model // 4

    def body(g_ref, c_ref, p_ref, qt_ref, wp_ref, wf_ref, bp_ref, bf_ref,
             o_ref):
        wf = wf_ref[...]
        wq2 = jnp.dot(qt_ref[...], wf[D_GATE:D_GATE + d4],
                      preferred_element_type=jnp.float32)
        wp2 = jnp.dot(wp_ref[...], wf[D_GATE + d4:],
                      preferred_element_type=jnp.float32)
        bias = jnp.dot(bp_ref[...], wf[D_GATE + d4:],
                       preferred_element_type=jnp.float32) + bf_ref[...]
        acc = jnp.dot(g_ref[...], wf[:D_GATE],
                      preferred_element_type=jnp.float32)
        acc = acc + jnp.dot(c_ref[...], wq2,
                            preferred_element_type=jnp.float32) * (1.0 / NQ)
        acc = acc + jnp.dot(p_ref[...], wp2,
                            preferred_element_type=jnp.float32)
        o_ref[...] = acc + bias

    return pl.pallas_call(
        body,
        grid=(m // tm,),
        in_specs=[
            pl.BlockSpec((tm, D_GATE), lambda i: (i, 0)),
            pl.BlockSpec((tm, NQ), lambda i: (i, 0)),
            pl.BlockSpec((tm, 8), lambda i: (i, 0)),
            pl.BlockSpec((NQ, d_model // 4), lambda i: (0, 0)),
            pl.BlockSpec((8, d_model // 4), lambda i: (0, 0)),
            pl.BlockSpec((d_model, d_model), lambda i: (0, 0)),
            pl.BlockSpec((1, d_model // 4), lambda i: (0, 0)),
            pl.BlockSpec((1, d_model), lambda i: (0, 0)),
        ],
        out_specs=pl.BlockSpec((tm, d_model), lambda i: (i, 0)),
        out_shape=jax.ShapeDtypeStruct((m, d_model), jnp.float32),
    )(gate_emb, counts, params, qubit_table, Wp, Wf, bp2, bf2)


def kernel(gates, qubits, parameters, gate_table, qubit_table, Wp, bp, Wf,
           bf):
    b, s = gates.shape
    m = b * s
    d_model = Wf.shape[0]

    gates_flat = gates.reshape(m).astype(jnp.int32)
    qubits_flat = qubits.reshape(m * NQ).astype(jnp.int32)
    params2 = parameters.reshape(m, parameters.shape[-1])

    gate_emb, counts_flat = _sc_gather_counts(gates_flat, qubits_flat,
                                              gate_table, m)
    counts2 = counts_flat.reshape(m, NQ)

    out = _tc_combine(gate_emb, counts2, params2, qubit_table, Wp,
                      bp.reshape(1, -1), Wf, bf.reshape(1, -1),
                      m, d_model, tm=1024)
    return out.reshape(b, s, d_model)


# speedup vs baseline: 2.8626x; 1.0972x over previous
import functools

import jax
import jax.numpy as jnp
from jax import lax
from jax.experimental import pallas as pl
from jax.experimental.pallas import tpu as pltpu
from jax.experimental.pallas import tpu_sc as plsc

NC, NS = 2, 16
NW = NC * NS
NQ = 32
D_GATE = 256
CHUNK = 128


def _sc_gather_counts(gates_flat, qubits_flat, gate_table, m):
    tpw = m // NW
    nchunk = tpw // CHUNK
    mesh = plsc.VectorSubcoreMesh(core_axis_name="c", subcore_axis_name="s")

    @functools.partial(
        pl.kernel,
        mesh=mesh,
        out_type=(
            jax.ShapeDtypeStruct((m, D_GATE), jnp.float32),
            jax.ShapeDtypeStruct((m * NQ,), jnp.float32),
        ),
        scratch_types=[
            pltpu.VMEM((CHUNK,), jnp.int32),
            pltpu.VMEM((CHUNK, D_GATE), jnp.float32),
            pltpu.VMEM((CHUNK * NQ,), jnp.int32),
            pltpu.VMEM((CHUNK * NQ,), jnp.float32),
            pltpu.SemaphoreType.DMA,
        ],
        compiler_params=pltpu.CompilerParams(needs_layout_passes=False),
    )
    def k(gates_hbm, qubits_hbm, table_hbm, emb_hbm, counts_hbm,
          idx_v, rows_v, qub_v, cnt_v, sem):
        wid = lax.axis_index("s") * NC + lax.axis_index("c")
        base0 = wid * tpw

        ones = jnp.ones((16,), jnp.float32)
        zeros = jnp.zeros((16,), jnp.float32)

        def chunk_body(ci, carry):
            base = base0 + ci * CHUNK
            pltpu.sync_copy(gates_hbm.at[pl.ds(base, CHUNK)], idx_v)
            gather = pltpu.async_copy(table_hbm.at[idx_v], rows_v, sem)
            pltpu.sync_copy(qubits_hbm.at[pl.ds(base * NQ, CHUNK * NQ)],
                            qub_v)

            def zero_body(j, c):
                cnt_v[pl.ds(j * 16, 16)] = zeros
                return c
            lax.fori_loop(0, CHUNK * NQ // 16, zero_body, None, unroll=8)

            def tok_body(t, c):
                b = t * NQ
                q0 = qub_v[pl.ds(b, 16)]
                q1 = qub_v[pl.ds(b + 16, 16)]
                plsc.addupdate_scatter(cnt_v, [q0 + b], ones)
                plsc.addupdate_scatter(cnt_v, [q1 + b], ones)
                return c
            lax.fori_loop(0, CHUNK, tok_body, None, unroll=4)

            gather.wait()
            pltpu.sync_copy(rows_v, emb_hbm.at[pl.ds(base, CHUNK)])
            pltpu.sync_copy(cnt_v,
                            counts_hbm.at[pl.ds(base * NQ, CHUNK * NQ)])
            return carry

        lax.fori_loop(0, nchunk, chunk_body, None)

    return k(gates_flat, qubits_flat, gate_table)


def _tc_combine(gate_emb, counts, params, qubit_table, Wp, bp2, Wf, bf2,
                m, d_model, tm):
    d4 = d_model // 4

    def body(g_ref, c_ref, p_ref, qt_ref, wp_ref, wf_ref, bp_ref, bf_ref,
             o_ref):
        wf = wf_ref[...]
        wq2 = jnp.dot(qt_ref[...], wf[D_GATE:D_GATE + d4],
                      preferred_element_type=jnp.float32)
        wp2 = jnp.dot(wp_ref[...], wf[D_GATE + d4:],
                      preferred_element_type=jnp.float32)
        bias = jnp.dot(bp_ref[...], wf[D_GATE + d4:],
                       preferred_element_type=jnp.float32) + bf_ref[...]
        acc = jnp.dot(g_ref[...], wf[:D_GATE],
                      preferred_element_type=jnp.float32)
        acc = acc + jnp.dot(c_ref[...], wq2,
                            preferred_element_type=jnp.float32) * (1.0 / NQ)
        acc = acc + jnp.dot(p_ref[...], wp2,
                            preferred_element_type=jnp.float32)
        o_ref[...] = acc + bias

    return pl.pallas_call(
        body,
        grid=(m // tm,),
        in_specs=[
            pl.BlockSpec((tm, D_GATE), lambda i: (i, 0)),
            pl.BlockSpec((tm, NQ), lambda i: (i, 0)),
            pl.BlockSpec((tm, 8), lambda i: (i, 0)),
            pl.BlockSpec((NQ, d_model // 4), lambda i: (0, 0)),
            pl.BlockSpec((8, d_model // 4), lambda i: (0, 0)),
            pl.BlockSpec((d_model, d_model), lambda i: (0, 0)),
            pl.BlockSpec((1, d_model // 4), lambda i: (0, 0)),
            pl.BlockSpec((1, d_model), lambda i: (0, 0)),
        ],
        out_specs=pl.BlockSpec((tm, d_model), lambda i: (i, 0)),
        out_shape=jax.ShapeDtypeStruct((m, d_model), jnp.float32),
    )(gate_emb, counts, params, qubit_table, Wp, Wf, bp2, bf2)


def kernel(gates, qubits, parameters, gate_table, qubit_table, Wp, bp, Wf,
           bf):
    b, s = gates.shape
    m = b * s
    d_model = Wf.shape[0]

    gates_flat = gates.reshape(m).astype(jnp.int32)
    qubits_flat = qubits.reshape(m * NQ).astype(jnp.int32)
    params2 = parameters.reshape(m, parameters.shape[-1])

    gate_emb, counts_flat = _sc_gather_counts(gates_flat, qubits_flat,
                                              gate_table, m)
    counts2 = counts_flat.reshape(m, NQ)

    out = _tc_combine(gate_emb, counts2, params2, qubit_table, Wp,
                      bp.reshape(1, -1), Wf, bf.reshape(1, -1),
                      m, d_model, tm=2048)
    return out.reshape(b, s, d_model)


# speedup vs baseline: 2.9843x; 1.0425x over previous
import functools

import jax
import jax.numpy as jnp
from jax import lax
from jax.experimental import pallas as pl
from jax.experimental.pallas import tpu as pltpu
from jax.experimental.pallas import tpu_sc as plsc

NC, NS = 2, 16
NW = NC * NS
NQ = 32
D_GATE = 256
CHUNK = 128


def _sc_gather_counts(gates_flat, qubits_flat, gate_table, m):
    tpw = m // NW
    nchunk = tpw // CHUNK
    mesh = plsc.VectorSubcoreMesh(core_axis_name="c", subcore_axis_name="s")

    @functools.partial(
        pl.kernel,
        mesh=mesh,
        out_type=(
            jax.ShapeDtypeStruct((m, D_GATE), jnp.float32),
            jax.ShapeDtypeStruct((m * NQ,), jnp.float32),
        ),
        scratch_types=[
            pltpu.VMEM((CHUNK,), jnp.int32),
            pltpu.VMEM((CHUNK, D_GATE), jnp.float32),
            pltpu.VMEM((CHUNK * NQ,), jnp.int32),
            pltpu.VMEM((CHUNK * NQ,), jnp.float32),
            pltpu.SemaphoreType.DMA,
        ],
        compiler_params=pltpu.CompilerParams(needs_layout_passes=False),
    )
    def k(gates_hbm, qubits_hbm, table_hbm, emb_hbm, counts_hbm,
          idx_v, rows_v, qub_v, cnt_v, sem):
        wid = lax.axis_index("s") * NC + lax.axis_index("c")
        base0 = wid * tpw

        ones = jnp.ones((16,), jnp.float32)
        zeros = jnp.zeros((16,), jnp.float32)

        def chunk_body(ci, carry):
            base = base0 + ci * CHUNK
            pltpu.sync_copy(gates_hbm.at[pl.ds(base, CHUNK)], idx_v)
            gather = pltpu.async_copy(table_hbm.at[idx_v], rows_v, sem)
            pltpu.sync_copy(qubits_hbm.at[pl.ds(base * NQ, CHUNK * NQ)],
                            qub_v)

            def zero_body(j, c):
                cnt_v[pl.ds(j * 16, 16)] = zeros
                return c
            lax.fori_loop(0, CHUNK * NQ // 16, zero_body, None, unroll=8)

            def tok_body(t, c):
                b = t * NQ
                q0 = qub_v[pl.ds(b, 16)]
                q1 = qub_v[pl.ds(b + 16, 16)]
                plsc.addupdate_scatter(cnt_v, [q0 + b], ones)
                plsc.addupdate_scatter(cnt_v, [q1 + b], ones)
                return c
            lax.fori_loop(0, CHUNK, tok_body, None, unroll=4)

            gather.wait()
            pltpu.sync_copy(rows_v, emb_hbm.at[pl.ds(base, CHUNK)])
            pltpu.sync_copy(cnt_v,
                            counts_hbm.at[pl.ds(base * NQ, CHUNK * NQ)])
            return carry

        lax.fori_loop(0, nchunk, chunk_body, None)

    return k(gates_flat, qubits_flat, gate_table)


def _tc_combine(gate_emb, counts, params, qubit_table, Wp, bp2, Wf, bf2,
                m, d_model, tm):
    d4 = d_model // 4

    def body(g_ref, c_ref, p_ref, qt_ref, wp_ref, wf_ref, bp_ref, bf_ref,
             o_ref):
        wf = wf_ref[...]
        wq2 = jnp.dot(qt_ref[...], wf[D_GATE:D_GATE + d4],
                      preferred_element_type=jnp.float32)
        wp2 = jnp.dot(wp_ref[...], wf[D_GATE + d4:],
                      preferred_element_type=jnp.float32)
        bias = jnp.dot(bp_ref[...], wf[D_GATE + d4:],
                       preferred_element_type=jnp.float32) + bf_ref[...]
        acc = jnp.dot(g_ref[...], wf[:D_GATE],
                      preferred_element_type=jnp.float32)
        acc = acc + jnp.dot(c_ref[...], wq2,
                            preferred_element_type=jnp.float32) * (1.0 / NQ)
        acc = acc + jnp.dot(p_ref[...], wp2,
                            preferred_element_type=jnp.float32)
        o_ref[...] = acc + bias

    return pl.pallas_call(
        body,
        grid=(m // tm,),
        in_specs=[
            pl.BlockSpec((tm, D_GATE), lambda i: (i, 0)),
            pl.BlockSpec((tm, NQ), lambda i: (i, 0)),
            pl.BlockSpec((tm, 8), lambda i: (i, 0)),
            pl.BlockSpec((NQ, d_model // 4), lambda i: (0, 0)),
            pl.BlockSpec((8, d_model // 4), lambda i: (0, 0)),
            pl.BlockSpec((d_model, d_model), lambda i: (0, 0)),
            pl.BlockSpec((1, d_model // 4), lambda i: (0, 0)),
            pl.BlockSpec((1, d_model), lambda i: (0, 0)),
        ],
        out_specs=pl.BlockSpec((tm, d_model), lambda i: (i, 0)),
        out_shape=jax.ShapeDtypeStruct((m, d_model), jnp.float32),
    )(gate_emb, counts, params, qubit_table, Wp, Wf, bp2, bf2)


def kernel(gates, qubits, parameters, gate_table, qubit_table, Wp, bp, Wf,
           bf):
    b, s = gates.shape
    m = b * s
    d_model = Wf.shape[0]

    gates_flat = gates.reshape(m).astype(jnp.int32)
    qubits_flat = qubits.reshape(m * NQ).astype(jnp.int32)
    params2 = parameters.reshape(m, parameters.shape[-1])

    gate_emb, counts_flat = _sc_gather_counts(gates_flat, qubits_flat,
                                              gate_table, m)
    counts2 = counts_flat.reshape(m, NQ)

    out = _tc_combine(gate_emb, counts2, params2, qubit_table, Wp,
                      bp.reshape(1, -1), Wf, bf.reshape(1, -1),
                      m, d_model, tm=4096)
    return out.reshape(b, s, d_model)
